# baseline (speedup 1.0000x reference)
import functools

import jax
import jax.numpy as jnp
from jax import lax
from jax.experimental import pallas as pl
from jax.experimental.pallas import tpu as pltpu
from jax.experimental.pallas import tpu_sc as plsc

N = 100000
E = 1600000
NUM_GRAPHS = 64
NC, NS, L = 2, 16, 16
NW = NC * NS
C = 2048
IPC = C // 128
CB = 1024
IPB = CB // 128
EP = 1638400
PAD = EP - E
CHUNKS = EP // C
GB = 5000
NB = N // GB
EB = 6400
NEG = -1e30


def _ale_body(c0_ref, c1_ref, c2_ref, c3_ref, moff_ref, v_ref,
              a0_ref, a1_ref, a2_ref, csum_ref, cmax_ref):
    i = pl.program_id(0)
    v = v_ref[...]
    cols = (c0_ref[...], c1_ref[...], c2_ref[...], c3_ref[...])
    moff = moff_ref[...]
    row = lax.broadcasted_iota(jnp.int32, (8, 128), 0)
    col = lax.broadcasted_iota(jnp.int32, (8, 128), 1)
    sums = []
    maxs = []
    for l, aref in enumerate((a0_ref, a1_ref, a2_ref)):
        al = (cols[0] * v[0, l] + cols[1] * v[1, l]
              + cols[2] * v[2, l] + cols[3] * v[3, l])
        aout = al + moff
        aref[...] = aout
        sums.append(jnp.sum(al))
        maxs.append(jnp.max(aout))
    upd = jnp.where(col == 0, sums[0],
                    jnp.where(col == 1, sums[1],
                              jnp.where(col == 2, sums[2], 0.0)))
    upd = jnp.where(row == 0, upd, 0.0)
    updm = jnp.where(col == 0, maxs[0],
                     jnp.where(col == 1, maxs[1],
                               jnp.where(col == 2, maxs[2], NEG)))

    @pl.when(i == 0)
    def _():
        csum_ref[...] = jnp.zeros_like(csum_ref)
        cmax_ref[...] = jnp.full_like(cmax_ref, NEG)

    csum_ref[...] += upd
    cmax_ref[...] = jnp.maximum(cmax_ref[...], updm)


EBA = 16384

_ale_call = pl.pallas_call(
    _ale_body,
    grid=(EP // EBA,),
    in_specs=[
        pl.BlockSpec((EBA,), lambda i: (i,)),
        pl.BlockSpec((EBA,), lambda i: (i,)),
        pl.BlockSpec((EBA,), lambda i: (i,)),
        pl.BlockSpec((EBA,), lambda i: (i,)),
        pl.BlockSpec((EBA,), lambda i: (i,)),
        pl.BlockSpec((4, 3), lambda i: (0, 0)),
    ],
    out_specs=[
        pl.BlockSpec((EBA,), lambda i: (i,)),
        pl.BlockSpec((EBA,), lambda i: (i,)),
        pl.BlockSpec((EBA,), lambda i: (i,)),
        pl.BlockSpec((8, 128), lambda i: (0, 0)),
        pl.BlockSpec((8, 128), lambda i: (0, 0)),
    ],
    out_shape=[
        jax.ShapeDtypeStruct((EP,), jnp.float32),
        jax.ShapeDtypeStruct((EP,), jnp.float32),
        jax.ShapeDtypeStruct((EP,), jnp.float32),
        jax.ShapeDtypeStruct((8, 128), jnp.float32),
        jax.ShapeDtypeStruct((8, 128), jnp.float32),
    ],
)


def _dense1_body(x_ref, W_ref, as_ref, ad_ref, c_ref,
                 h_ref, als_ref, ald_ref, lal_ref, lmax_ref):
    i = pl.program_id(0)
    h = jnp.dot(x_ref[...], W_ref[...], preferred_element_type=jnp.float32)
    h_ref[...] = h
    als = jnp.sum(h * as_ref[...][None, :], axis=1)
    ald = jnp.sum(h * ad_ref[...][None, :], axis=1)
    als_ref[...] = als[None, None, :]
    ald_ref[...] = ald[None, None, :]
    lal = als + ald + c_ref[0, 0]
    lal = jnp.where(lal > 0, lal, 0.2 * lal)
    lal_ref[...] = lal[None, None, :]
    col = lax.broadcasted_iota(jnp.int32, (8, 128), 1)
    upd = jnp.where(col == 0, jnp.max(als),
                    jnp.where(col == 1, jnp.max(ald),
                              jnp.where(col == 2, jnp.max(lal), NEG)))

    @pl.when(i == 0)
    def _():
        lmax_ref[...] = jnp.full_like(lmax_ref, NEG)

    lmax_ref[...] = jnp.maximum(lmax_ref[...], upd)


_dense1_call = pl.pallas_call(
    _dense1_body,
    grid=(NB,),
    in_specs=[
        pl.BlockSpec((GB, 128), lambda i: (i, 0)),
        pl.BlockSpec((128, 16), lambda i: (0, 0)),
        pl.BlockSpec((16,), lambda i: (0,)),
        pl.BlockSpec((16,), lambda i: (0,)),
        pl.BlockSpec((1, 1), lambda i: (0, 0)),
    ],
    out_specs=[
        pl.BlockSpec((GB, 16), lambda i: (i, 0)),
        pl.BlockSpec((1, 1, GB), lambda i: (i, 0, 0)),
        pl.BlockSpec((1, 1, GB), lambda i: (i, 0, 0)),
        pl.BlockSpec((1, 1, GB), lambda i: (i, 0, 0)),
        pl.BlockSpec((8, 128), lambda i: (0, 0)),
    ],
    out_shape=[
        jax.ShapeDtypeStruct((N, 16), jnp.float32),
        jax.ShapeDtypeStruct((NB, 1, GB), jnp.float32),
        jax.ShapeDtypeStruct((NB, 1, GB), jnp.float32),
        jax.ShapeDtypeStruct((NB, 1, GB), jnp.float32),
        jax.ShapeDtypeStruct((8, 128), jnp.float32),
    ],
)


def _epilogue(mode, numv, denv, hv, pl_, b):
    dent = denv[0, 0, 0] + denv[1, 0, 0] + pl_ + 1e-30
    dri = 1.0 / dent
    if mode == "edges":
        numt = numv[0] + numv[1] + pl_[:, None] * hv
    else:
        numt = jnp.concatenate(
            [numv[0] + pl_[:, None] * hv[0], numv[1] + pl_[:, None] * hv[1]],
            axis=1)
    return numt * dri[:, None] + b[None, :]


def _dense_mid_body(mode, num_ref, den_ref, h_ref, lal_ref, M_ref, b_ref,
                    W_ref, as_ref, ad_ref, c_ref,
                    hout_ref, als_ref, ald_ref, lalo_ref, lmax_ref):
    i = pl.program_id(0)
    pl_ = jnp.exp(lal_ref[...][0, 0] - M_ref[0, 0])
    hf = _epilogue(mode, num_ref[...], den_ref[...], h_ref[...], pl_, b_ref[...])
    hf = jnp.maximum(hf, 0.0)
    h2 = jnp.dot(hf, W_ref[...], preferred_element_type=jnp.float32)
    hout_ref[0] = h2[:, :16]
    hout_ref[1] = h2[:, 16:]
    als = jnp.sum(h2 * as_ref[...][None, :], axis=1)
    ald = jnp.sum(h2 * ad_ref[...][None, :], axis=1)
    als_ref[...] = als[None, None, :]
    ald_ref[...] = ald[None, None, :]
    lal = als + ald + c_ref[0, 0]
    lal = jnp.where(lal > 0, lal, 0.2 * lal)
    lalo_ref[...] = lal[None, None, :]
    col = lax.broadcasted_iota(jnp.int32, (8, 128), 1)
    upd = jnp.where(col == 0, jnp.max(als),
                    jnp.where(col == 1, jnp.max(ald),
                              jnp.where(col == 2, jnp.max(lal), NEG)))

    @pl.when(i == 0)
    def _():
        lmax_ref[...] = jnp.full_like(lmax_ref, NEG)

    lmax_ref[...] = jnp.maximum(lmax_ref[...], upd)


def _make_dense_mid(mode, d_in_prev):
    if mode == "edges":
        h_spec = pl.BlockSpec((GB, 16), lambda i: (i, 0))
        h_shape_prev = None
        b_len = 16
    else:
        h_spec = pl.BlockSpec((2, GB, 16), lambda i: (0, i, 0))
        b_len = 32
    return pl.pallas_call(
        functools.partial(_dense_mid_body, mode),
        grid=(NB,),
        in_specs=[
            pl.BlockSpec((2, GB, 16), lambda i: (0, i, 0)),
            pl.BlockSpec((2, 1, 1, GB), lambda i: (0, i, 0, 0)),
            h_spec,
            pl.BlockSpec((1, 1, GB), lambda i: (i, 0, 0)),
            pl.BlockSpec((1, 1), lambda i: (0, 0)),
            pl.BlockSpec((b_len,), lambda i: (0,)),
            pl.BlockSpec((d_in_prev, 32), lambda i: (0, 0)),
            pl.BlockSpec((32,), lambda i: (0,)),
            pl.BlockSpec((32,), lambda i: (0,)),
            pl.BlockSpec((1, 1), lambda i: (0, 0)),
        ],
        out_specs=[
            pl.BlockSpec((2, GB, 16), lambda i: (0, i, 0)),
            pl.BlockSpec((1, 1, GB), lambda i: (i, 0, 0)),
            pl.BlockSpec((1, 1, GB), lambda i: (i, 0, 0)),
            pl.BlockSpec((1, 1, GB), lambda i: (i, 0, 0)),
            pl.BlockSpec((8, 128), lambda i: (0, 0)),
        ],
        out_shape=[
            jax.ShapeDtypeStruct((2, N, 16), jnp.float32),
            jax.ShapeDtypeStruct((NB, 1, GB), jnp.float32),
            jax.ShapeDtypeStruct((NB, 1, GB), jnp.float32),
            jax.ShapeDtypeStruct((NB, 1, GB), jnp.float32),
            jax.ShapeDtypeStruct((8, 128), jnp.float32),
        ],
    )


_dense2_call = _make_dense_mid("edges", 16)
_dense3_call = _make_dense_mid("channels", 32)


def _pool_body(num_ref, den_ref, h_ref, lal_ref, M_ref, b_ref, batch_ref, g_ref):
    i = pl.program_id(0)
    pl_ = jnp.exp(lal_ref[...][0, 0] - M_ref[0, 0])
    h3 = _epilogue("channels", num_ref[...], den_ref[...], h_ref[...], pl_,
                   b_ref[...])
    bt = batch_ref[...][0]
    ohT = (lax.broadcasted_iota(jnp.int32, (NUM_GRAPHS, 1), 0) == bt
           ).astype(jnp.float32)
    gp = jnp.dot(ohT, h3, preferred_element_type=jnp.float32)

    @pl.when(i == 0)
    def _():
        g_ref[...] = jnp.zeros_like(g_ref)

    g_ref[...] += gp


_pool_call = pl.pallas_call(
    _pool_body,
    grid=(NB,),
    in_specs=[
        pl.BlockSpec((2, GB, 16), lambda i: (0, i, 0)),
        pl.BlockSpec((2, 1, 1, GB), lambda i: (0, i, 0, 0)),
        pl.BlockSpec((2, GB, 16), lambda i: (0, i, 0)),
        pl.BlockSpec((1, 1, GB), lambda i: (i, 0, 0)),
        pl.BlockSpec((1, 1), lambda i: (0, 0)),
        pl.BlockSpec((32,), lambda i: (0,)),
        pl.BlockSpec((1, 1, GB), lambda i: (i, 0, 0)),
    ],
    out_specs=pl.BlockSpec((NUM_GRAPHS, 32), lambda i: (0, 0)),
    out_shape=jax.ShapeDtypeStruct((NUM_GRAPHS, 32), jnp.float32),
)


def _fc_body(g_ref, w_ref, b_ref, o_ref):
    o_ref[...] = jnp.dot(g_ref[...], w_ref[...],
                         preferred_element_type=jnp.float32) + b_ref[...][None, :]


_fc_call = pl.pallas_call(
    _fc_body,
    out_shape=jax.ShapeDtypeStruct((NUM_GRAPHS, 1024), jnp.float32),
)


_MESH = plsc.VectorSubcoreMesh(core_axis_name="c", subcore_axis_name="s",
                               num_cores=NC, num_subcores=NS)


def _sc_aggr_body(split_edges, row_off_mult,
                  src2d, dst2d, ale1, als_t, ald_t, m16, htab, znum, zden,
                  num_out, den_out,
                  idx_sA, idx_sB, idx_dA, idx_dB, alevA, alevB, gsv, pv,
                  rows, m16v, num_s, den_s,
                  sem_in, sem_a, sem_g, sem_s, sem_d):
    c = lax.axis_index("c")
    s = lax.axis_index("s")
    wid = c * NS + s
    npt = 6256
    npt_last = N - (NS - 1) * npt

    @pl.when(s < NS - 1)
    def _():
        off = pl.multiple_of(s * npt, 8)
        pltpu.sync_copy(znum.at[pl.ds(off, npt)], num_s.at[pl.ds(off, npt)])

    @pl.when(s == NS - 1)
    def _():
        pltpu.sync_copy(znum.at[pl.ds((NS - 1) * npt, npt_last)],
                        num_s.at[pl.ds((NS - 1) * npt, npt_last)])
        pltpu.sync_copy(zden, den_s)

    pltpu.sync_copy(m16, m16v)
    plsc.subcore_barrier()

    nch = 50 if split_edges else 100
    bufs = ((idx_sA, idx_dA, alevA), (idx_sB, idx_dB, alevB))

    def chunk_k(j):
        if split_edges:
            return j * NW + wid
        return s * 100 + j

    def fire_inputs(j, bsel):
        k = chunk_k(j)
        kr = pl.multiple_of(k * IPB, 8)
        ke = pl.multiple_of(k * CB, 8)
        i_s, i_d, alv = bufs[bsel]
        pltpu.async_copy(src2d.at[pl.ds(kr, IPB)], i_s, sem_in)
        pltpu.async_copy(dst2d.at[pl.ds(kr, IPB)], i_d, sem_in)
        pltpu.async_copy(ale1.at[pl.ds(ke, CB)], alv, sem_in)

    def wait_inputs(j, bsel):
        k = chunk_k(j)
        kr = pl.multiple_of(k * IPB, 8)
        ke = pl.multiple_of(k * CB, 8)
        i_s, i_d, alv = bufs[bsel]
        pltpu.make_async_copy(src2d.at[pl.ds(kr, IPB)], i_s, sem_in).wait()
        pltpu.make_async_copy(dst2d.at[pl.ds(kr, IPB)], i_d, sem_in).wait()
        pltpu.make_async_copy(ale1.at[pl.ds(ke, CB)], alv, sem_in).wait()

    def drain_scatters(i_d, prev_den):
        for j2 in range(IPB):
            pltpu.make_async_copy(rows.at[pl.ds(j2 * 128, 128)],
                                  num_s.at[i_d.at[j2]], sem_s).wait()

        @pl.when(prev_den)
        def _():
            for j2 in range(IPB):
                pltpu.make_async_copy(pv.at[pl.ds(j2 * 128, 128)],
                                      den_s.at[i_d.at[j2]], sem_d).wait()

    fire_inputs(0, 0)

    def chunk(j, carry):
        k = chunk_k(j)
        for b in range(2):
            @pl.when((j % 2) == b)
            def _(b=b):
                i_s, i_d, alv = bufs[b]
                o_s, o_d, _o = bufs[1 - b]
                @pl.when(j > 0)
                def _():
                    if split_edges:
                        drain_scatters(o_d, j > 0)
                    else:
                        drain_scatters(o_d, ((k - 1) % 2) == c)
                wait_inputs(j, b)
                ga_ = [pltpu.async_copy(als_t.at[i_s.at[j2]],
                                        gsv.at[pl.ds(j2 * 128, 128)], sem_a)
                       for j2 in range(IPB)]
                gb_ = [pltpu.async_copy(ald_t.at[i_d.at[j2]],
                                        pv.at[pl.ds(j2 * 128, 128)], sem_a)
                       for j2 in range(IPB)]
                @pl.when(j + 1 < nch)
                def _():
                    fire_inputs(j + 1, 1 - b)
                for d in ga_:
                    d.wait()
                for d in gb_:
                    d.wait()
                if row_off_mult:
                    roff = c * row_off_mult
                    for j2 in range(IPB):
                        for l2 in range(128 // L):
                            sl2 = (j2, pl.ds(l2 * L, L))
                            i_s[sl2] = i_s[sl2] + roff
                gd_ = [pltpu.async_copy(htab.at[i_s.at[j2]],
                                       rows.at[pl.ds(j2 * 128, 128)], sem_g)
                       for j2 in range(IPB)]
                mv = m16v[...]
                for g in range(CB // L):
                    sl = pl.ds(g * L, L)
                    a = gsv[sl] + pv[sl] + alv[sl]
                    a = jnp.where(a > 0, a, a * 0.2)
                    pv[sl] = jnp.exp(a - mv)
                for d in gd_:
                    d.wait()

                def ebody(i2, cc):
                    base = i2 * L
                    pvec = pv[pl.ds(base, L)]
                    for j2 in range(L):
                        e = base + j2
                        rows[e, :] = rows[e, :] * pvec[j2]
                    return cc

                lax.fori_loop(0, CB // L, ebody, 0)
                for j2 in range(IPB):
                    pltpu.async_copy(rows.at[pl.ds(j2 * 128, 128)],
                                     num_s.at[i_d.at[j2]], sem_s, add=True)
                if split_edges:
                    for j2 in range(IPB):
                        pltpu.async_copy(pv.at[pl.ds(j2 * 128, 128)],
                                         den_s.at[i_d.at[j2]], sem_d, add=True)
                else:
                    @pl.when((k % 2) == c)
                    def _():
                        for j2 in range(IPB):
                            pltpu.async_copy(pv.at[pl.ds(j2 * 128, 128)],
                                             den_s.at[i_d.at[j2]], sem_d,
                                             add=True)
        return carry

    lax.fori_loop(0, nch, chunk, 0)
    klast = chunk_k(nch - 1)
    for b in range(2):
        @pl.when(((nch - 1) % 2) == b)
        def _(b=b):
            if split_edges:
                drain_scatters(bufs[b][1], True)
            else:
                drain_scatters(bufs[b][1], (klast % 2) == c)
    plsc.subcore_barrier()

    @pl.when(s < NS - 1)
    def _():
        off = pl.multiple_of(s * npt, 8)
        offo = pl.multiple_of(c * N + s * npt, 8)
        pltpu.sync_copy(num_s.at[pl.ds(off, npt)],
                        num_out.at[pl.ds(offo, npt)])

    @pl.when(s == NS - 1)
    def _():
        offo = pl.multiple_of(c * N + (NS - 1) * npt, 8)
        pltpu.sync_copy(num_s.at[pl.ds((NS - 1) * npt, npt_last)],
                        num_out.at[pl.ds(offo, npt_last)])
        pltpu.sync_copy(den_s, den_out.at[pl.ds(pl.multiple_of(c * N, 8), N)])


def _make_aggr(split_edges, row_off_mult):
    return pl.kernel(
        functools.partial(_sc_aggr_body, split_edges, row_off_mult),
        out_type=[
            jax.ShapeDtypeStruct((2 * N, 16), jnp.float32),
            jax.ShapeDtypeStruct((2 * N,), jnp.float32),
        ],
        mesh=_MESH,
        scratch_types=[
            pltpu.VMEM((IPB, 128), jnp.int32),
            pltpu.VMEM((IPB, 128), jnp.int32),
            pltpu.VMEM((IPB, 128), jnp.int32),
            pltpu.VMEM((IPB, 128), jnp.int32),
            pltpu.VMEM((CB,), jnp.float32),
            pltpu.VMEM((CB,), jnp.float32),
            pltpu.VMEM((CB,), jnp.float32),
            pltpu.VMEM((CB,), jnp.float32),
            pltpu.VMEM((CB, 16), jnp.float32),
            pltpu.VMEM((L,), jnp.float32),
            pltpu.VMEM_SHARED((N, 16), jnp.float32),
            pltpu.VMEM_SHARED((N,), jnp.float32),
            pltpu.SemaphoreType.DMA,
            pltpu.SemaphoreType.DMA,
            pltpu.SemaphoreType.DMA,
            pltpu.SemaphoreType.DMA,
            pltpu.SemaphoreType.DMA,
        ],
        compiler_params=pltpu.CompilerParams(use_tc_tiling_on_sc=False),
    )


_aggr_edges_call = _make_aggr(True, 0)
_aggr_channels_call = _make_aggr(False, N)


def kernel(x, edge_index, edge_attr, batch, W1, as1, ad1, We1, ae1, b1,
           W2, as2, ad2, We2, ae2, b2, W3, as3, ad3, We3, ae3, b3, Wfc, bfc):
    f32 = jnp.float32
    src, dst = edge_index[0], edge_index[1]
    zi = jnp.zeros((PAD,), jnp.int32)
    src2d = jnp.concatenate([src, zi]).reshape(EP // 128, 128)
    dst2d = jnp.concatenate([dst, zi]).reshape(EP // 128, 128)
    v3 = jnp.stack([We1 @ ae1, We2 @ ae2, We3 @ ae3], axis=1)
    zf = jnp.zeros((PAD,), f32)
    colp = [jnp.concatenate([edge_attr[:, j], zf]) for j in range(4)]
    moff = jnp.concatenate([jnp.zeros((E,), f32), jnp.full((PAD,), NEG, f32)])
    ale0, ale1, ale2, csum, cmax = _ale_call(
        colp[0], colp[1], colp[2], colp[3], moff, v3)
    cvec = csum[0, :3] / E
    alep = [ale0, ale1, ale2]
    znum = jnp.zeros((N, 16), f32)
    zden = jnp.zeros((N,), f32)
    batch3 = batch.reshape(NB, 1, GB)

    def softmax_shift(stats, l):
        s_ub = stats[0, 0] + stats[0, 1] + cmax[0, l]
        m_ub = jnp.where(s_ub > 0, s_ub, 0.2 * s_ub)
        return jnp.maximum(m_ub, stats[0, 2])

    h1, als1, ald1, lal1, st1 = _dense1_call(
        x, W1, as1, ad1, cvec[0].reshape(1, 1))
    M1 = softmax_shift(st1, 0)
    num1, den1 = _aggr_edges_call(src2d, dst2d, alep[0], als1.reshape(N),
                                  ald1.reshape(N), jnp.full((L,), M1, f32),
                                  h1, znum, zden)

    h2, als2, ald2, lal2, st2 = _dense2_call(
        num1.reshape(2, N, 16), den1.reshape(2, NB, 1, GB), h1, lal1,
        M1.reshape(1, 1), b1, W2, as2, ad2, cvec[1].reshape(1, 1))
    M2 = softmax_shift(st2, 1)
    num2, den2 = _aggr_channels_call(src2d, dst2d, alep[1], als2.reshape(N),
                                     ald2.reshape(N), jnp.full((L,), M2, f32),
                                     h2.reshape(2 * N, 16), znum, zden)

    h3, als3, ald3, lal3, st3 = _dense3_call(
        num2.reshape(2, N, 16), den2.reshape(2, NB, 1, GB), h2, lal2,
        M2.reshape(1, 1), b2, W3, as3, ad3, cvec[2].reshape(1, 1))
    M3 = softmax_shift(st3, 2)
    num3, den3 = _aggr_channels_call(src2d, dst2d, alep[2], als3.reshape(N),
                                     ald3.reshape(N), jnp.full((L,), M3, f32),
                                     h3.reshape(2 * N, 16), znum, zden)

    g = _pool_call(num3.reshape(2, N, 16), den3.reshape(2, NB, 1, GB), h3,
                   lal3, M3.reshape(1, 1), b3, batch3)
    return _fc_call(g, Wfc, bfc)


# speedup vs baseline: 1.2150x; 1.2150x over previous
import functools

import jax
import jax.numpy as jnp
from jax import lax
from jax.experimental import pallas as pl
from jax.experimental.pallas import tpu as pltpu
from jax.experimental.pallas import tpu_sc as plsc

N = 100000
E = 1600000
NUM_GRAPHS = 64
NC, NS, L = 2, 16, 16
NW = NC * NS
C = 2048
IPC = C // 128
CB = 1024
IPB = CB // 128
EP = 1638400
PAD = EP - E
CHUNKS = EP // C
GB = 5000
NB = N // GB
EB = 6400
NEG = -1e30


def _ale_body(c0_ref, c1_ref, c2_ref, c3_ref, moff_ref, v_ref,
              a0_ref, a1_ref, a2_ref, csum_ref, cmax_ref):
    i = pl.program_id(0)
    v = v_ref[...]
    cols = (c0_ref[...], c1_ref[...], c2_ref[...], c3_ref[...])
    moff = moff_ref[...]
    row = lax.broadcasted_iota(jnp.int32, (8, 128), 0)
    col = lax.broadcasted_iota(jnp.int32, (8, 128), 1)
    sums = []
    maxs = []
    for l, aref in enumerate((a0_ref, a1_ref, a2_ref)):
        al = (cols[0] * v[0, l] + cols[1] * v[1, l]
              + cols[2] * v[2, l] + cols[3] * v[3, l])
        aout = al + moff
        aref[...] = aout
        sums.append(jnp.sum(al))
        maxs.append(jnp.max(aout))
    upd = jnp.where(col == 0, sums[0],
                    jnp.where(col == 1, sums[1],
                              jnp.where(col == 2, sums[2], 0.0)))
    upd = jnp.where(row == 0, upd, 0.0)
    updm = jnp.where(col == 0, maxs[0],
                     jnp.where(col == 1, maxs[1],
                               jnp.where(col == 2, maxs[2], NEG)))

    @pl.when(i == 0)
    def _():
        csum_ref[...] = jnp.zeros_like(csum_ref)
        cmax_ref[...] = jnp.full_like(cmax_ref, NEG)

    csum_ref[...] += upd
    cmax_ref[...] = jnp.maximum(cmax_ref[...], updm)


EBA = 16384

_ale_call = pl.pallas_call(
    _ale_body,
    grid=(EP // EBA,),
    in_specs=[
        pl.BlockSpec((EBA,), lambda i: (i,)),
        pl.BlockSpec((EBA,), lambda i: (i,)),
        pl.BlockSpec((EBA,), lambda i: (i,)),
        pl.BlockSpec((EBA,), lambda i: (i,)),
        pl.BlockSpec((EBA,), lambda i: (i,)),
        pl.BlockSpec((4, 3), lambda i: (0, 0)),
    ],
    out_specs=[
        pl.BlockSpec((EBA,), lambda i: (i,)),
        pl.BlockSpec((EBA,), lambda i: (i,)),
        pl.BlockSpec((EBA,), lambda i: (i,)),
        pl.BlockSpec((8, 128), lambda i: (0, 0)),
        pl.BlockSpec((8, 128), lambda i: (0, 0)),
    ],
    out_shape=[
        jax.ShapeDtypeStruct((EP,), jnp.float32),
        jax.ShapeDtypeStruct((EP,), jnp.float32),
        jax.ShapeDtypeStruct((EP,), jnp.float32),
        jax.ShapeDtypeStruct((8, 128), jnp.float32),
        jax.ShapeDtypeStruct((8, 128), jnp.float32),
    ],
)


def _dense1_body(x_ref, W_ref, as_ref, ad_ref, c_ref,
                 h_ref, als_ref, ald_ref, lal_ref, lmax_ref):
    i = pl.program_id(0)
    h = jnp.dot(x_ref[...], W_ref[...], preferred_element_type=jnp.float32)
    h_ref[...] = h
    als = jnp.sum(h * as_ref[...][None, :], axis=1)
    ald = jnp.sum(h * ad_ref[...][None, :], axis=1)
    als_ref[...] = als[None, None, :]
    ald_ref[...] = ald[None, None, :]
    lal = als + ald + c_ref[0, 0]
    lal = jnp.where(lal > 0, lal, 0.2 * lal)
    lal_ref[...] = lal[None, None, :]
    col = lax.broadcasted_iota(jnp.int32, (8, 128), 1)
    upd = jnp.where(col == 0, jnp.max(als),
                    jnp.where(col == 1, jnp.max(ald),
                              jnp.where(col == 2, jnp.max(lal), NEG)))

    @pl.when(i == 0)
    def _():
        lmax_ref[...] = jnp.full_like(lmax_ref, NEG)

    lmax_ref[...] = jnp.maximum(lmax_ref[...], upd)


_dense1_call = pl.pallas_call(
    _dense1_body,
    grid=(NB,),
    in_specs=[
        pl.BlockSpec((GB, 128), lambda i: (i, 0)),
        pl.BlockSpec((128, 16), lambda i: (0, 0)),
        pl.BlockSpec((16,), lambda i: (0,)),
        pl.BlockSpec((16,), lambda i: (0,)),
        pl.BlockSpec((1, 1), lambda i: (0, 0)),
    ],
    out_specs=[
        pl.BlockSpec((GB, 16), lambda i: (i, 0)),
        pl.BlockSpec((1, 1, GB), lambda i: (i, 0, 0)),
        pl.BlockSpec((1, 1, GB), lambda i: (i, 0, 0)),
        pl.BlockSpec((1, 1, GB), lambda i: (i, 0, 0)),
        pl.BlockSpec((8, 128), lambda i: (0, 0)),
    ],
    out_shape=[
        jax.ShapeDtypeStruct((N, 16), jnp.float32),
        jax.ShapeDtypeStruct((NB, 1, GB), jnp.float32),
        jax.ShapeDtypeStruct((NB, 1, GB), jnp.float32),
        jax.ShapeDtypeStruct((NB, 1, GB), jnp.float32),
        jax.ShapeDtypeStruct((8, 128), jnp.float32),
    ],
)


def _epilogue_halves(mode, numv, denv, hv, pl_, b):
    dent = denv[0, 0, 0] + denv[1, 0, 0] + pl_ + 1e-30
    dri = (1.0 / dent)[:, None]
    if mode == "edges":
        numt = (numv[0] + numv[1] + pl_[:, None] * hv) * dri + b[None, :16]
        return numt, None
    lo = (numv[0] + pl_[:, None] * hv[0]) * dri + b[None, :16]
    hi = (numv[1] + pl_[:, None] * hv[1]) * dri + b[None, 16:]
    return lo, hi


def _dense_mid_body(mode, num_ref, den_ref, h_ref, lal_ref, M_ref, b_ref,
                    W_ref, as_ref, ad_ref, c_ref,
                    hout_ref, als_ref, ald_ref, lalo_ref, lmax_ref):
    i = pl.program_id(0)
    pl_ = jnp.exp(lal_ref[...][0, 0] - M_ref[0, 0])
    W = W_ref[...]
    asv = as_ref[...]
    adv = ad_ref[...]
    if mode == "edges":
        hf, _ = _epilogue_halves(mode, num_ref[...], den_ref[...], h_ref[...],
                                 pl_, b_ref[...])
        hf = jnp.maximum(hf, 0.0)
        h2lo = jnp.dot(hf, W[:, :16], preferred_element_type=jnp.float32)
        h2hi = jnp.dot(hf, W[:, 16:], preferred_element_type=jnp.float32)
    else:
        lo, hi = _epilogue_halves(mode, num_ref[...], den_ref[...], h_ref[...],
                                  pl_, b_ref[...])
        lo = jnp.maximum(lo, 0.0)
        hi = jnp.maximum(hi, 0.0)
        h2lo = (jnp.dot(lo, W[:16, :16], preferred_element_type=jnp.float32)
                + jnp.dot(hi, W[16:, :16], preferred_element_type=jnp.float32))
        h2hi = (jnp.dot(lo, W[:16, 16:], preferred_element_type=jnp.float32)
                + jnp.dot(hi, W[16:, 16:], preferred_element_type=jnp.float32))
    hout_ref[0] = h2lo
    hout_ref[1] = h2hi
    als = (jnp.sum(h2lo * asv[None, :16], axis=1)
           + jnp.sum(h2hi * asv[None, 16:], axis=1))
    ald = (jnp.sum(h2lo * adv[None, :16], axis=1)
           + jnp.sum(h2hi * adv[None, 16:], axis=1))
    als_ref[...] = als[None, None, :]
    ald_ref[...] = ald[None, None, :]
    lal = als + ald + c_ref[0, 0]
    lal = jnp.where(lal > 0, lal, 0.2 * lal)
    lalo_ref[...] = lal[None, None, :]
    col = lax.broadcasted_iota(jnp.int32, (8, 128), 1)
    upd = jnp.where(col == 0, jnp.max(als),
                    jnp.where(col == 1, jnp.max(ald),
                              jnp.where(col == 2, jnp.max(lal), NEG)))

    @pl.when(i == 0)
    def _():
        lmax_ref[...] = jnp.full_like(lmax_ref, NEG)

    lmax_ref[...] = jnp.maximum(lmax_ref[...], upd)


def _make_dense_mid(mode, d_in_prev):
    if mode == "edges":
        h_spec = pl.BlockSpec((GB, 16), lambda i: (i, 0))
        h_shape_prev = None
        b_len = 16
    else:
        h_spec = pl.BlockSpec((2, GB, 16), lambda i: (0, i, 0))
        b_len = 32
    return pl.pallas_call(
        functools.partial(_dense_mid_body, mode),
        grid=(NB,),
        in_specs=[
            pl.BlockSpec((2, GB, 16), lambda i: (0, i, 0)),
            pl.BlockSpec((2, 1, 1, GB), lambda i: (0, i, 0, 0)),
            h_spec,
            pl.BlockSpec((1, 1, GB), lambda i: (i, 0, 0)),
            pl.BlockSpec((1, 1), lambda i: (0, 0)),
            pl.BlockSpec((b_len,), lambda i: (0,)),
            pl.BlockSpec((d_in_prev, 32), lambda i: (0, 0)),
            pl.BlockSpec((32,), lambda i: (0,)),
            pl.BlockSpec((32,), lambda i: (0,)),
            pl.BlockSpec((1, 1), lambda i: (0, 0)),
        ],
        out_specs=[
            pl.BlockSpec((2, GB, 16), lambda i: (0, i, 0)),
            pl.BlockSpec((1, 1, GB), lambda i: (i, 0, 0)),
            pl.BlockSpec((1, 1, GB), lambda i: (i, 0, 0)),
            pl.BlockSpec((1, 1, GB), lambda i: (i, 0, 0)),
            pl.BlockSpec((8, 128), lambda i: (0, 0)),
        ],
        out_shape=[
            jax.ShapeDtypeStruct((2, N, 16), jnp.float32),
            jax.ShapeDtypeStruct((NB, 1, GB), jnp.float32),
            jax.ShapeDtypeStruct((NB, 1, GB), jnp.float32),
            jax.ShapeDtypeStruct((NB, 1, GB), jnp.float32),
            jax.ShapeDtypeStruct((8, 128), jnp.float32),
        ],
    )


_dense2_call = _make_dense_mid("edges", 16)
_dense3_call = _make_dense_mid("channels", 32)


def _pool_body(num_ref, den_ref, h_ref, lal_ref, M_ref, b_ref, batch_ref,
               g_ref):
    i = pl.program_id(0)
    pl_ = jnp.exp(lal_ref[...][0, 0] - M_ref[0, 0])
    lo, hi = _epilogue_halves("channels", num_ref[...], den_ref[...],
                              h_ref[...], pl_, b_ref[...])
    bt = batch_ref[...][0]
    ohT = (lax.broadcasted_iota(jnp.int32, (NUM_GRAPHS, 1), 0) == bt
           ).astype(jnp.float32)
    glo = jnp.dot(ohT, lo, preferred_element_type=jnp.float32)
    ghi = jnp.dot(ohT, hi, preferred_element_type=jnp.float32)

    @pl.when(i == 0)
    def _():
        g_ref[...] = jnp.zeros_like(g_ref)

    g_ref[0] += glo
    g_ref[1] += ghi


_pool_call = pl.pallas_call(
    _pool_body,
    grid=(NB,),
    in_specs=[
        pl.BlockSpec((2, GB, 16), lambda i: (0, i, 0)),
        pl.BlockSpec((2, 1, 1, GB), lambda i: (0, i, 0, 0)),
        pl.BlockSpec((2, GB, 16), lambda i: (0, i, 0)),
        pl.BlockSpec((1, 1, GB), lambda i: (i, 0, 0)),
        pl.BlockSpec((1, 1), lambda i: (0, 0)),
        pl.BlockSpec((32,), lambda i: (0,)),
        pl.BlockSpec((1, 1, GB), lambda i: (i, 0, 0)),
    ],
    out_specs=pl.BlockSpec((2, NUM_GRAPHS, 16), lambda i: (0, 0, 0)),
    out_shape=jax.ShapeDtypeStruct((2, NUM_GRAPHS, 16), jnp.float32),
)


def _fc_body(g_ref, w_ref, b_ref, o_ref):
    g = g_ref[...]
    w = w_ref[...]
    o_ref[...] = (jnp.dot(g[0], w[:16, :], preferred_element_type=jnp.float32)
                  + jnp.dot(g[1], w[16:, :],
                            preferred_element_type=jnp.float32)
                  + b_ref[...][None, :])


_fc_call = pl.pallas_call(
    _fc_body,
    out_shape=jax.ShapeDtypeStruct((NUM_GRAPHS, 1024), jnp.float32),
)


_MESH = plsc.VectorSubcoreMesh(core_axis_name="c", subcore_axis_name="s",
                               num_cores=NC, num_subcores=NS)


def _sc_aggr_body(split_edges, row_off_mult,
                  src2d, dst2d, ale1, als_t, ald_t, m16, htab, znum, zden,
                  num_out, den_out,
                  idx_s, idx_a, idx_d, alev, gsv, gdv, pv, rows, m16v,
                  num_s, den_s, sem_a, sem_g, sem_s, sem_d):
    c = lax.axis_index("c")
    s = lax.axis_index("s")
    wid = c * NS + s
    npt = 6256
    npt_last = N - (NS - 1) * npt

    @pl.when(s < NS - 1)
    def _():
        off = pl.multiple_of(s * npt, 8)
        pltpu.sync_copy(znum.at[pl.ds(off, npt)], num_s.at[pl.ds(off, npt)])

    @pl.when(s == NS - 1)
    def _():
        pltpu.sync_copy(znum.at[pl.ds((NS - 1) * npt, npt_last)],
                        num_s.at[pl.ds((NS - 1) * npt, npt_last)])
        pltpu.sync_copy(zden, den_s)

    pltpu.sync_copy(m16, m16v)
    plsc.subcore_barrier()

    nch = 50 if split_edges else 100

    def chunk(j, carry):
        if split_edges:
            k = j * NW + wid
        else:
            k = s * 100 + j
        kr = pl.multiple_of(k * IPB, 8)
        ke = pl.multiple_of(k * CB, 8)
        in_ = [pltpu.async_copy(src2d.at[pl.ds(kr, IPB)], idx_s, sem_g),
               pltpu.async_copy(dst2d.at[pl.ds(kr, IPB)], idx_d, sem_g),
               pltpu.async_copy(ale1.at[pl.ds(ke, CB)], alev, sem_g)]
        for d in in_:
            d.wait()
        ga_ = [pltpu.async_copy(als_t.at[idx_s.at[j2]],
                                gsv.at[pl.ds(j2 * 128, 128)], sem_a)
               for j2 in range(IPB)]
        gb_ = [pltpu.async_copy(ald_t.at[idx_d.at[j2]],
                                gdv.at[pl.ds(j2 * 128, 128)], sem_a)
               for j2 in range(IPB)]
        if row_off_mult:
            roff = c * row_off_mult
            for j2 in range(IPB):
                for l2 in range(128 // L):
                    sl2 = (j2, pl.ds(l2 * L, L))
                    idx_a[sl2] = idx_s[sl2] + roff
            src_idx = idx_a
        else:
            src_idx = idx_s
        gd_ = [pltpu.async_copy(htab.at[src_idx.at[j2]],
                                rows.at[pl.ds(j2 * 128, 128)], sem_g)
               for j2 in range(IPB)]
        for d in ga_:
            d.wait()
        for d in gb_:
            d.wait()
        mv = m16v[...]
        for g in range(CB // L):
            sl = pl.ds(g * L, L)
            a = gsv[sl] + gdv[sl] + alev[sl]
            a = jnp.where(a > 0, a, a * 0.2)
            pv[sl] = jnp.exp(a - mv)
        for d in gd_:
            d.wait()

        def ebody(i2, cc):
            base = i2 * L
            pvec = pv[pl.ds(base, L)]
            for j2 in range(L):
                e = base + j2
                rows[e, :] = rows[e, :] * pvec[j2]
            return cc

        lax.fori_loop(0, CB // L, ebody, 0)
        sc_ = [pltpu.async_copy(rows.at[pl.ds(j2 * 128, 128)],
                                num_s.at[idx_d.at[j2]], sem_s, add=True)
               for j2 in range(IPB)]
        for d in sc_:
            d.wait()

        def dden():
            dd_ = [pltpu.async_copy(pv.at[pl.ds(j2 * 128, 128)],
                                    den_s.at[idx_d.at[j2]], sem_d, add=True)
                   for j2 in range(IPB)]
            for d in dd_:
                d.wait()

        if split_edges:
            dden()
        else:
            @pl.when((k % 2) == c)
            def _():
                dden()

        return carry

    lax.fori_loop(0, nch, chunk, 0)
    plsc.subcore_barrier()

    @pl.when(s < NS - 1)
    def _():
        off = pl.multiple_of(s * npt, 8)
        offo = pl.multiple_of(c * N + s * npt, 8)
        pltpu.sync_copy(num_s.at[pl.ds(off, npt)],
                        num_out.at[pl.ds(offo, npt)])

    @pl.when(s == NS - 1)
    def _():
        offo = pl.multiple_of(c * N + (NS - 1) * npt, 8)
        pltpu.sync_copy(num_s.at[pl.ds((NS - 1) * npt, npt_last)],
                        num_out.at[pl.ds(offo, npt_last)])
        pltpu.sync_copy(den_s, den_out.at[pl.ds(pl.multiple_of(c * N, 8), N)])


def _make_aggr(split_edges, row_off_mult):
    return pl.kernel(
        functools.partial(_sc_aggr_body, split_edges, row_off_mult),
        out_type=[
            jax.ShapeDtypeStruct((2 * N, 16), jnp.float32),
            jax.ShapeDtypeStruct((2 * N,), jnp.float32),
        ],
        mesh=_MESH,
        scratch_types=[
            pltpu.VMEM((IPB, 128), jnp.int32),
            pltpu.VMEM((IPB, 128), jnp.int32),
            pltpu.VMEM((IPB, 128), jnp.int32),
            pltpu.VMEM((CB,), jnp.float32),
            pltpu.VMEM((CB,), jnp.float32),
            pltpu.VMEM((CB,), jnp.float32),
            pltpu.VMEM((CB,), jnp.float32),
            pltpu.VMEM((CB, 16), jnp.float32),
            pltpu.VMEM((L,), jnp.float32),
            pltpu.VMEM_SHARED((N, 16), jnp.float32),
            pltpu.VMEM_SHARED((N,), jnp.float32),
            pltpu.SemaphoreType.DMA,
            pltpu.SemaphoreType.DMA,
            pltpu.SemaphoreType.DMA,
            pltpu.SemaphoreType.DMA,
        ],
        compiler_params=pltpu.CompilerParams(use_tc_tiling_on_sc=False),
    )


_aggr_edges_call = _make_aggr(True, 0)
_aggr_channels_call = _make_aggr(False, N)


def kernel(x, edge_index, edge_attr, batch, W1, as1, ad1, We1, ae1, b1,
           W2, as2, ad2, We2, ae2, b2, W3, as3, ad3, We3, ae3, b3, Wfc, bfc):
    f32 = jnp.float32
    src, dst = edge_index[0], edge_index[1]
    zi = jnp.zeros((PAD,), jnp.int32)
    src2d = jnp.concatenate([src, zi]).reshape(EP // 128, 128)
    dst2d = jnp.concatenate([dst, zi]).reshape(EP // 128, 128)
    v3 = jnp.stack([We1 @ ae1, We2 @ ae2, We3 @ ae3], axis=1)
    zf = jnp.zeros((PAD,), f32)
    colp = [jnp.concatenate([edge_attr[:, j], zf]) for j in range(4)]
    moff = jnp.concatenate([jnp.zeros((E,), f32), jnp.full((PAD,), NEG, f32)])
    ale0, ale1, ale2, csum, cmax = _ale_call(
        colp[0], colp[1], colp[2], colp[3], moff, v3)
    cvec = csum[0, :3] / E
    alep = [ale0, ale1, ale2]
    znum = jnp.zeros((N, 16), f32)
    zden = jnp.zeros((N,), f32)
    batch3 = batch.reshape(NB, 1, GB)

    def softmax_shift(stats, l):
        s_ub = stats[0, 0] + stats[0, 1] + cmax[0, l]
        m_ub = jnp.where(s_ub > 0, s_ub, 0.2 * s_ub)
        return jnp.maximum(m_ub, stats[0, 2])

    h1, als1, ald1, lal1, st1 = _dense1_call(
        x, W1, as1, ad1, cvec[0].reshape(1, 1))
    M1 = softmax_shift(st1, 0)
    num1, den1 = _aggr_edges_call(src2d, dst2d, alep[0], als1.reshape(N),
                                  ald1.reshape(N), jnp.full((L,), M1, f32),
                                  h1, znum, zden)

    h2, als2, ald2, lal2, st2 = _dense2_call(
        num1.reshape(2, N, 16), den1.reshape(2, NB, 1, GB), h1, lal1,
        M1.reshape(1, 1), b1, W2, as2, ad2, cvec[1].reshape(1, 1))
    M2 = softmax_shift(st2, 1)
    num2, den2 = _aggr_channels_call(src2d, dst2d, alep[1], als2.reshape(N),
                                     ald2.reshape(N), jnp.full((L,), M2, f32),
                                     h2.reshape(2 * N, 16), znum, zden)

    h3, als3, ald3, lal3, st3 = _dense3_call(
        num2.reshape(2, N, 16), den2.reshape(2, NB, 1, GB), h2, lal2,
        M2.reshape(1, 1), b2, W3, as3, ad3, cvec[2].reshape(1, 1))
    M3 = softmax_shift(st3, 2)
    num3, den3 = _aggr_channels_call(src2d, dst2d, alep[2], als3.reshape(N),
                                     ald3.reshape(N), jnp.full((L,), M3, f32),
                                     h3.reshape(2 * N, 16), znum, zden)

    g = _pool_call(num3.reshape(2, N, 16), den3.reshape(2, NB, 1, GB), h3,
                   lal3, M3.reshape(1, 1), b3, batch3)
    return _fc_call(g, Wfc, bfc)


# speedup vs baseline: 1.2312x; 1.0133x over previous
import functools

import jax
import jax.numpy as jnp
from jax import lax
from jax.experimental import pallas as pl
from jax.experimental.pallas import tpu as pltpu
from jax.experimental.pallas import tpu_sc as plsc

N = 100000
E = 1600000
NUM_GRAPHS = 64
NC, NS, L = 2, 16, 16
NW = NC * NS
C = 2048
IPC = C // 128
CB = 1024
IPB = CB // 128
EP = 1638400
PAD = EP - E
CHUNKS = EP // C
GB = 5000
NB = N // GB
EB = 6400
NEG = -1e30


def _ale_body(c0_ref, c1_ref, c2_ref, c3_ref, moff_ref, v_ref,
              a0_ref, a1_ref, a2_ref, csum_ref, cmax_ref):
    i = pl.program_id(0)
    v = v_ref[...]
    cols = (c0_ref[...], c1_ref[...], c2_ref[...], c3_ref[...])
    moff = moff_ref[...]
    row = lax.broadcasted_iota(jnp.int32, (8, 128), 0)
    col = lax.broadcasted_iota(jnp.int32, (8, 128), 1)
    sums = []
    maxs = []
    for l, aref in enumerate((a0_ref, a1_ref, a2_ref)):
        al = (cols[0] * v[0, l] + cols[1] * v[1, l]
              + cols[2] * v[2, l] + cols[3] * v[3, l])
        aout = al + moff
        aref[...] = aout
        sums.append(jnp.sum(al))
        maxs.append(jnp.max(aout))
    upd = jnp.where(col == 0, sums[0],
                    jnp.where(col == 1, sums[1],
                              jnp.where(col == 2, sums[2], 0.0)))
    upd = jnp.where(row == 0, upd, 0.0)
    updm = jnp.where(col == 0, maxs[0],
                     jnp.where(col == 1, maxs[1],
                               jnp.where(col == 2, maxs[2], NEG)))

    @pl.when(i == 0)
    def _():
        csum_ref[...] = jnp.zeros_like(csum_ref)
        cmax_ref[...] = jnp.full_like(cmax_ref, NEG)

    csum_ref[...] += upd
    cmax_ref[...] = jnp.maximum(cmax_ref[...], updm)


EBA = 16384

_ale_call = pl.pallas_call(
    _ale_body,
    grid=(EP // EBA,),
    in_specs=[
        pl.BlockSpec((EBA,), lambda i: (i,)),
        pl.BlockSpec((EBA,), lambda i: (i,)),
        pl.BlockSpec((EBA,), lambda i: (i,)),
        pl.BlockSpec((EBA,), lambda i: (i,)),
        pl.BlockSpec((EBA,), lambda i: (i,)),
        pl.BlockSpec((4, 3), lambda i: (0, 0)),
    ],
    out_specs=[
        pl.BlockSpec((EBA,), lambda i: (i,)),
        pl.BlockSpec((EBA,), lambda i: (i,)),
        pl.BlockSpec((EBA,), lambda i: (i,)),
        pl.BlockSpec((8, 128), lambda i: (0, 0)),
        pl.BlockSpec((8, 128), lambda i: (0, 0)),
    ],
    out_shape=[
        jax.ShapeDtypeStruct((EP,), jnp.float32),
        jax.ShapeDtypeStruct((EP,), jnp.float32),
        jax.ShapeDtypeStruct((EP,), jnp.float32),
        jax.ShapeDtypeStruct((8, 128), jnp.float32),
        jax.ShapeDtypeStruct((8, 128), jnp.float32),
    ],
)


def _dense1_body(x_ref, W_ref, as_ref, ad_ref, c_ref,
                 h_ref, als_ref, ald_ref, lal_ref, lmax_ref):
    i = pl.program_id(0)
    h = jnp.dot(x_ref[...], W_ref[...], preferred_element_type=jnp.float32)
    h_ref[...] = h
    als = jnp.sum(h * as_ref[...][None, :], axis=1)
    ald = jnp.sum(h * ad_ref[...][None, :], axis=1)
    als_ref[...] = als[None, None, :]
    ald_ref[...] = ald[None, None, :]
    lal = als + ald + c_ref[0, 0]
    lal = jnp.where(lal > 0, lal, 0.2 * lal)
    lal_ref[...] = lal[None, None, :]
    col = lax.broadcasted_iota(jnp.int32, (8, 128), 1)
    upd = jnp.where(col == 0, jnp.max(als),
                    jnp.where(col == 1, jnp.max(ald),
                              jnp.where(col == 2, jnp.max(lal), NEG)))

    @pl.when(i == 0)
    def _():
        lmax_ref[...] = jnp.full_like(lmax_ref, NEG)

    lmax_ref[...] = jnp.maximum(lmax_ref[...], upd)


_dense1_call = pl.pallas_call(
    _dense1_body,
    grid=(NB,),
    in_specs=[
        pl.BlockSpec((GB, 128), lambda i: (i, 0)),
        pl.BlockSpec((128, 16), lambda i: (0, 0)),
        pl.BlockSpec((16,), lambda i: (0,)),
        pl.BlockSpec((16,), lambda i: (0,)),
        pl.BlockSpec((1, 1), lambda i: (0, 0)),
    ],
    out_specs=[
        pl.BlockSpec((GB, 16), lambda i: (i, 0)),
        pl.BlockSpec((1, 1, GB), lambda i: (i, 0, 0)),
        pl.BlockSpec((1, 1, GB), lambda i: (i, 0, 0)),
        pl.BlockSpec((1, 1, GB), lambda i: (i, 0, 0)),
        pl.BlockSpec((8, 128), lambda i: (0, 0)),
    ],
    out_shape=[
        jax.ShapeDtypeStruct((N, 16), jnp.float32),
        jax.ShapeDtypeStruct((NB, 1, GB), jnp.float32),
        jax.ShapeDtypeStruct((NB, 1, GB), jnp.float32),
        jax.ShapeDtypeStruct((NB, 1, GB), jnp.float32),
        jax.ShapeDtypeStruct((8, 128), jnp.float32),
    ],
)


def _epilogue_halves(mode, numv, denv, hv, pl_, b):
    dent = denv[0, 0, 0] + denv[1, 0, 0] + pl_ + 1e-30
    dri = (1.0 / dent)[:, None]
    if mode == "edges":
        numt = (numv[0] + numv[1] + pl_[:, None] * hv) * dri + b[None, :16]
        return numt, None
    lo = (numv[0] + pl_[:, None] * hv[0]) * dri + b[None, :16]
    hi = (numv[1] + pl_[:, None] * hv[1]) * dri + b[None, 16:]
    return lo, hi


def _dense_mid_body(mode, num_ref, den_ref, h_ref, lal_ref, M_ref, b_ref,
                    W_ref, as_ref, ad_ref, c_ref,
                    hout_ref, als_ref, ald_ref, lalo_ref, lmax_ref):
    i = pl.program_id(0)
    pl_ = jnp.exp(lal_ref[...][0, 0] - M_ref[0, 0])
    lo, hi = _epilogue_halves(mode, num_ref[...], den_ref[...], h_ref[...],
                              pl_, b_ref[...])
    W = W_ref[...]
    if mode == "edges":
        hf = jnp.maximum(lo, 0.0)
        h2 = jnp.dot(hf, W, preferred_element_type=jnp.float32)
    else:
        hf = jnp.maximum(jnp.concatenate([lo, hi], axis=1), 0.0)
        h2 = jnp.dot(hf, W, preferred_element_type=jnp.float32)
    hout_ref[0] = h2[:, :16]
    hout_ref[1] = h2[:, 16:]
    als = jnp.sum(h2 * as_ref[...][None, :], axis=1)
    ald = jnp.sum(h2 * ad_ref[...][None, :], axis=1)
    als_ref[...] = als[None, None, :]
    ald_ref[...] = ald[None, None, :]
    lal = als + ald + c_ref[0, 0]
    lal = jnp.where(lal > 0, lal, 0.2 * lal)
    lalo_ref[...] = lal[None, None, :]
    col = lax.broadcasted_iota(jnp.int32, (8, 128), 1)
    upd = jnp.where(col == 0, jnp.max(als),
                    jnp.where(col == 1, jnp.max(ald),
                              jnp.where(col == 2, jnp.max(lal), NEG)))

    @pl.when(i == 0)
    def _():
        lmax_ref[...] = jnp.full_like(lmax_ref, NEG)

    lmax_ref[...] = jnp.maximum(lmax_ref[...], upd)


def _make_dense_mid(mode, d_in_prev):
    if mode == "edges":
        h_spec = pl.BlockSpec((GB, 16), lambda i: (i, 0))
        h_shape_prev = None
        b_len = 16
    else:
        h_spec = pl.BlockSpec((2, GB, 16), lambda i: (0, i, 0))
        b_len = 32
    return pl.pallas_call(
        functools.partial(_dense_mid_body, mode),
        grid=(NB,),
        in_specs=[
            pl.BlockSpec((2, GB, 16), lambda i: (0, i, 0)),
            pl.BlockSpec((2, 1, 1, GB), lambda i: (0, i, 0, 0)),
            h_spec,
            pl.BlockSpec((1, 1, GB), lambda i: (i, 0, 0)),
            pl.BlockSpec((1, 1), lambda i: (0, 0)),
            pl.BlockSpec((b_len,), lambda i: (0,)),
            pl.BlockSpec((d_in_prev, 32), lambda i: (0, 0)),
            pl.BlockSpec((32,), lambda i: (0,)),
            pl.BlockSpec((32,), lambda i: (0,)),
            pl.BlockSpec((1, 1), lambda i: (0, 0)),
        ],
        out_specs=[
            pl.BlockSpec((2, GB, 16), lambda i: (0, i, 0)),
            pl.BlockSpec((1, 1, GB), lambda i: (i, 0, 0)),
            pl.BlockSpec((1, 1, GB), lambda i: (i, 0, 0)),
            pl.BlockSpec((1, 1, GB), lambda i: (i, 0, 0)),
            pl.BlockSpec((8, 128), lambda i: (0, 0)),
        ],
        out_shape=[
            jax.ShapeDtypeStruct((2, N, 16), jnp.float32),
            jax.ShapeDtypeStruct((NB, 1, GB), jnp.float32),
            jax.ShapeDtypeStruct((NB, 1, GB), jnp.float32),
            jax.ShapeDtypeStruct((NB, 1, GB), jnp.float32),
            jax.ShapeDtypeStruct((8, 128), jnp.float32),
        ],
    )


_dense2_call = _make_dense_mid("edges", 16)
_dense3_call = _make_dense_mid("channels", 32)


def _pool_body(num_ref, den_ref, h_ref, lal_ref, M_ref, b_ref, batch_ref,
               g_ref):
    i = pl.program_id(0)
    pl_ = jnp.exp(lal_ref[...][0, 0] - M_ref[0, 0])
    lo, hi = _epilogue_halves("channels", num_ref[...], den_ref[...],
                              h_ref[...], pl_, b_ref[...])
    bt = batch_ref[...][0]
    ohT = (lax.broadcasted_iota(jnp.int32, (NUM_GRAPHS, 1), 0) == bt
           ).astype(jnp.float32)
    glo = jnp.dot(ohT, lo, preferred_element_type=jnp.float32)
    ghi = jnp.dot(ohT, hi, preferred_element_type=jnp.float32)

    @pl.when(i == 0)
    def _():
        g_ref[...] = jnp.zeros_like(g_ref)

    g_ref[0] += glo
    g_ref[1] += ghi


_pool_call = pl.pallas_call(
    _pool_body,
    grid=(NB,),
    in_specs=[
        pl.BlockSpec((2, GB, 16), lambda i: (0, i, 0)),
        pl.BlockSpec((2, 1, 1, GB), lambda i: (0, i, 0, 0)),
        pl.BlockSpec((2, GB, 16), lambda i: (0, i, 0)),
        pl.BlockSpec((1, 1, GB), lambda i: (i, 0, 0)),
        pl.BlockSpec((1, 1), lambda i: (0, 0)),
        pl.BlockSpec((32,), lambda i: (0,)),
        pl.BlockSpec((1, 1, GB), lambda i: (i, 0, 0)),
    ],
    out_specs=pl.BlockSpec((2, NUM_GRAPHS, 16), lambda i: (0, 0, 0)),
    out_shape=jax.ShapeDtypeStruct((2, NUM_GRAPHS, 16), jnp.float32),
)


def _fc_body(g_ref, w_ref, b_ref, o_ref):
    g = g_ref[...]
    w = w_ref[...]
    o_ref[...] = (jnp.dot(g[0], w[:16, :], preferred_element_type=jnp.float32)
                  + jnp.dot(g[1], w[16:, :],
                            preferred_element_type=jnp.float32)
                  + b_ref[...][None, :])


_fc_call = pl.pallas_call(
    _fc_body,
    out_shape=jax.ShapeDtypeStruct((NUM_GRAPHS, 1024), jnp.float32),
)


_MESH = plsc.VectorSubcoreMesh(core_axis_name="c", subcore_axis_name="s",
                               num_cores=NC, num_subcores=NS)


def _sc_aggr_body(split_edges, row_off_mult,
                  src2d, dst2d, ale1, als_t, ald_t, m16, htab, znum, zden,
                  num_out, den_out,
                  idx_s, idx_a, idx_d, alev, gsv, gdv, pv, rows, m16v,
                  num_s, den_s, sem_a, sem_g, sem_s, sem_d):
    c = lax.axis_index("c")
    s = lax.axis_index("s")
    wid = c * NS + s
    npt = 6256
    npt_last = N - (NS - 1) * npt

    @pl.when(s < NS - 1)
    def _():
        off = pl.multiple_of(s * npt, 8)
        pltpu.sync_copy(znum.at[pl.ds(off, npt)], num_s.at[pl.ds(off, npt)])

    @pl.when(s == NS - 1)
    def _():
        pltpu.sync_copy(znum.at[pl.ds((NS - 1) * npt, npt_last)],
                        num_s.at[pl.ds((NS - 1) * npt, npt_last)])
        pltpu.sync_copy(zden, den_s)

    pltpu.sync_copy(m16, m16v)
    plsc.subcore_barrier()

    nch = 50 if split_edges else 100

    def chunk(j, carry):
        if split_edges:
            k = j * NW + wid
        else:
            k = s * 100 + j
        kr = pl.multiple_of(k * IPB, 8)
        ke = pl.multiple_of(k * CB, 8)
        in_ = [pltpu.async_copy(src2d.at[pl.ds(kr, IPB)], idx_s, sem_g),
               pltpu.async_copy(dst2d.at[pl.ds(kr, IPB)], idx_d, sem_g),
               pltpu.async_copy(ale1.at[pl.ds(ke, CB)], alev, sem_g)]
        for d in in_:
            d.wait()
        ga_ = [pltpu.async_copy(als_t.at[idx_s.at[j2]],
                                gsv.at[pl.ds(j2 * 128, 128)], sem_a)
               for j2 in range(IPB)]
        gb_ = [pltpu.async_copy(ald_t.at[idx_d.at[j2]],
                                gdv.at[pl.ds(j2 * 128, 128)], sem_a)
               for j2 in range(IPB)]
        if row_off_mult:
            roff = c * row_off_mult
            for j2 in range(IPB):
                for l2 in range(128 // L):
                    sl2 = (j2, pl.ds(l2 * L, L))
                    idx_a[sl2] = idx_s[sl2] + roff
            src_idx = idx_a
        else:
            src_idx = idx_s
        gd_ = [pltpu.async_copy(htab.at[src_idx.at[j2]],
                                rows.at[pl.ds(j2 * 128, 128)], sem_g)
               for j2 in range(IPB)]
        for d in ga_:
            d.wait()
        for d in gb_:
            d.wait()
        mv = m16v[...]
        for g in range(CB // L):
            sl = pl.ds(g * L, L)
            a = gsv[sl] + gdv[sl] + alev[sl]
            a = jnp.where(a > 0, a, a * 0.2)
            pv[sl] = jnp.exp(a - mv)
        for d in gd_:
            d.wait()

        def ebody(i2, cc):
            base = i2 * L
            pvec = pv[pl.ds(base, L)]
            for j2 in range(L):
                e = base + j2
                rows[e, :] = rows[e, :] * pvec[j2]
            return cc

        lax.fori_loop(0, CB // L, ebody, 0)
        sc_ = [pltpu.async_copy(rows.at[pl.ds(j2 * 128, 128)],
                                num_s.at[idx_d.at[j2]], sem_s, add=True)
               for j2 in range(IPB)]
        for d in sc_:
            d.wait()

        def dden():
            dd_ = [pltpu.async_copy(pv.at[pl.ds(j2 * 128, 128)],
                                    den_s.at[idx_d.at[j2]], sem_d, add=True)
                   for j2 in range(IPB)]
            for d in dd_:
                d.wait()

        if split_edges:
            dden()
        else:
            @pl.when((k % 2) == c)
            def _():
                dden()

        return carry

    lax.fori_loop(0, nch, chunk, 0)
    plsc.subcore_barrier()

    @pl.when(s < NS - 1)
    def _():
        off = pl.multiple_of(s * npt, 8)
        offo = pl.multiple_of(c * N + s * npt, 8)
        pltpu.sync_copy(num_s.at[pl.ds(off, npt)],
                        num_out.at[pl.ds(offo, npt)])

    @pl.when(s == NS - 1)
    def _():
        offo = pl.multiple_of(c * N + (NS - 1) * npt, 8)
        pltpu.sync_copy(num_s.at[pl.ds((NS - 1) * npt, npt_last)],
                        num_out.at[pl.ds(offo, npt_last)])
        pltpu.sync_copy(den_s, den_out.at[pl.ds(pl.multiple_of(c * N, 8), N)])


def _make_aggr(split_edges, row_off_mult):
    return pl.kernel(
        functools.partial(_sc_aggr_body, split_edges, row_off_mult),
        out_type=[
            jax.ShapeDtypeStruct((2 * N, 16), jnp.float32),
            jax.ShapeDtypeStruct((2 * N,), jnp.float32),
        ],
        mesh=_MESH,
        scratch_types=[
            pltpu.VMEM((IPB, 128), jnp.int32),
            pltpu.VMEM((IPB, 128), jnp.int32),
            pltpu.VMEM((IPB, 128), jnp.int32),
            pltpu.VMEM((CB,), jnp.float32),
            pltpu.VMEM((CB,), jnp.float32),
            pltpu.VMEM((CB,), jnp.float32),
            pltpu.VMEM((CB,), jnp.float32),
            pltpu.VMEM((CB, 16), jnp.float32),
            pltpu.VMEM((L,), jnp.float32),
            pltpu.VMEM_SHARED((N, 16), jnp.float32),
            pltpu.VMEM_SHARED((N,), jnp.float32),
            pltpu.SemaphoreType.DMA,
            pltpu.SemaphoreType.DMA,
            pltpu.SemaphoreType.DMA,
            pltpu.SemaphoreType.DMA,
        ],
        compiler_params=pltpu.CompilerParams(use_tc_tiling_on_sc=False),
    )


_aggr_edges_call = _make_aggr(True, 0)
_aggr_channels_call = _make_aggr(False, N)


def kernel(x, edge_index, edge_attr, batch, W1, as1, ad1, We1, ae1, b1,
           W2, as2, ad2, We2, ae2, b2, W3, as3, ad3, We3, ae3, b3, Wfc, bfc):
    f32 = jnp.float32
    src, dst = edge_index[0], edge_index[1]
    zi = jnp.zeros((PAD,), jnp.int32)
    src2d = jnp.concatenate([src, zi]).reshape(EP // 128, 128)
    dst2d = jnp.concatenate([dst, zi]).reshape(EP // 128, 128)
    v3 = jnp.stack([We1 @ ae1, We2 @ ae2, We3 @ ae3], axis=1)
    zf = jnp.zeros((PAD,), f32)
    colp = [jnp.concatenate([edge_attr[:, j], zf]) for j in range(4)]
    moff = jnp.concatenate([jnp.zeros((E,), f32), jnp.full((PAD,), NEG, f32)])
    ale0, ale1, ale2, csum, cmax = _ale_call(
        colp[0], colp[1], colp[2], colp[3], moff, v3)
    cvec = csum[0, :3] / E
    alep = [ale0, ale1, ale2]
    znum = jnp.zeros((N, 16), f32)
    zden = jnp.zeros((N,), f32)
    batch3 = batch.reshape(NB, 1, GB)

    def softmax_shift(stats, l):
        s_ub = stats[0, 0] + stats[0, 1] + cmax[0, l]
        m_ub = jnp.where(s_ub > 0, s_ub, 0.2 * s_ub)
        return jnp.maximum(m_ub, stats[0, 2])

    h1, als1, ald1, lal1, st1 = _dense1_call(
        x, W1, as1, ad1, cvec[0].reshape(1, 1))
    M1 = softmax_shift(st1, 0)
    num1, den1 = _aggr_edges_call(src2d, dst2d, alep[0], als1.reshape(N),
                                  ald1.reshape(N), jnp.full((L,), M1, f32),
                                  h1, znum, zden)

    h2, als2, ald2, lal2, st2 = _dense2_call(
        num1.reshape(2, N, 16), den1.reshape(2, NB, 1, GB), h1, lal1,
        M1.reshape(1, 1), b1, W2, as2, ad2, cvec[1].reshape(1, 1))
    M2 = softmax_shift(st2, 1)
    num2, den2 = _aggr_channels_call(src2d, dst2d, alep[1], als2.reshape(N),
                                     ald2.reshape(N), jnp.full((L,), M2, f32),
                                     h2.reshape(2 * N, 16), znum, zden)

    h3, als3, ald3, lal3, st3 = _dense3_call(
        num2.reshape(2, N, 16), den2.reshape(2, NB, 1, GB), h2, lal2,
        M2.reshape(1, 1), b2, W3, as3, ad3, cvec[2].reshape(1, 1))
    M3 = softmax_shift(st3, 2)
    num3, den3 = _aggr_channels_call(src2d, dst2d, alep[2], als3.reshape(N),
                                     ald3.reshape(N), jnp.full((L,), M3, f32),
                                     h3.reshape(2 * N, 16), znum, zden)

    g = _pool_call(num3.reshape(2, N, 16), den3.reshape(2, NB, 1, GB), h3,
                   lal3, M3.reshape(1, 1), b3, batch3)
    return _fc_call(g, Wfc, bfc)
